# baseline (speedup 1.0000x reference)
import functools

import jax
import jax.numpy as jnp
from jax import lax
from jax.experimental import pallas as pl
from jax.experimental.pallas import tpu as pltpu
from jax.experimental.pallas import tpu_sc as plsc

N = 65536
C = 1000
H = 512
L = 16
NC, NS = 2, 16
NW = NC * NS
ROWS_PW = N // NW
R = 32
CHUNKS = ROWS_PW // R
NFULL = C // L
NDUP = L - (C - NFULL * L)
TAIL = C - L

BLOCK_MLP = 2048


def _sc_energy_body(x_hbm, w_hbm, out_hbm, wbuf, xbuf, obuf):
    wid = lax.axis_index("s") * NC + lax.axis_index("c")
    base_row = wid * ROWS_PW

    pltpu.sync_copy(w_hbm, wbuf)

    def relu_body(j, carry):
        off = pl.multiple_of(j * L, L)
        wbuf[pl.ds(off, L)] = jnp.maximum(wbuf[pl.ds(off, L)], 0.0)
        return carry

    lax.fori_loop(0, NFULL, relu_body, 0)
    lane = lax.iota(jnp.int32, L)
    wt = jnp.where(lane < NDUP, 0.0, jnp.maximum(wbuf[pl.ds(TAIL, L)], 0.0))

    def chunk_body(ci, carry):
        row0 = ci * R
        pltpu.sync_copy(x_hbm.at[pl.ds(base_row + row0, R), :], xbuf)

        def row_body(r, carry2):
            def mx(j, mv):
                off = pl.multiple_of(j * L, L)
                return jnp.maximum(mv, xbuf[r, pl.ds(off, L)])

            mv = lax.fori_loop(0, NFULL, mx,
                               jnp.full((L,), -jnp.inf, jnp.float32))
            mv = jnp.maximum(mv, xbuf[r, pl.ds(TAIL, L)])

            def sm(j, sv):
                off = pl.multiple_of(j * L, L)
                return sv + (jnp.exp(xbuf[r, pl.ds(off, L)] - mv)
                             * wbuf[pl.ds(off, L)])

            sv = lax.fori_loop(0, NFULL, sm, jnp.zeros((L,), jnp.float32))
            sv = sv + jnp.exp(xbuf[r, pl.ds(TAIL, L)] - mv) * wt
            obuf[r, pl.ds(0, L)] = mv
            obuf[r, pl.ds(L, L)] = sv
            return carry2

        lax.fori_loop(0, R, row_body, 0)
        pltpu.sync_copy(obuf, out_hbm.at[pl.ds(base_row + row0, R), :])
        return carry

    lax.fori_loop(0, CHUNKS, chunk_body, 0)


def _sc_energy(cls_logits, w):
    mesh = plsc.VectorSubcoreMesh(core_axis_name="c", subcore_axis_name="s")
    kern = functools.partial(
        pl.kernel,
        mesh=mesh,
        out_type=jax.ShapeDtypeStruct((N, 2 * L), jnp.float32),
        scratch_types=[
            pltpu.VMEM((C,), jnp.float32),
            pltpu.VMEM((R, C), jnp.float32),
            pltpu.VMEM((R, 2 * L), jnp.float32),
        ],
    )(_sc_energy_body)
    return kern(cls_logits, w)


def _mlp_body(ms_ref, w1_ref, b1_ref, w2_ref, b2_ref, out_ref):
    mv = ms_ref[:, :L]
    sv = ms_ref[:, L:]
    m = jnp.max(mv, axis=1, keepdims=True)
    s = jnp.sum(sv * jnp.exp(mv - m), axis=1, keepdims=True)
    e = m + jnp.log(s)
    h = jax.nn.relu(e * w1_ref[...] + b1_ref[...])
    d = jnp.sum(h * w2_ref[...], axis=1, keepdims=True) + b2_ref[0, 0]
    out_ref[...] = jax.nn.sigmoid(d)


def _tc_mlp(ms, W1, b1, W2, b2):
    small = lambda shape: pl.BlockSpec(shape, lambda i: (0, 0))
    return pl.pallas_call(
        _mlp_body,
        grid=(N // BLOCK_MLP,),
        in_specs=[
            pl.BlockSpec((BLOCK_MLP, 2 * L), lambda i: (i, 0)),
            small((1, H)), small((1, H)), small((1, H)), small((1, 1)),
        ],
        out_specs=pl.BlockSpec((BLOCK_MLP, 1), lambda i: (i, 0)),
        out_shape=jax.ShapeDtypeStruct((N, 1), jnp.float32),
    )(ms, W1.reshape(1, H), b1.reshape(1, H), W2.reshape(1, H),
      b2.reshape(1, 1))


def kernel(cls_logits, energy_score_weights, W1, b1, W2, b2):
    ms = _sc_energy(cls_logits, energy_score_weights.reshape(C))
    return _tc_mlp(ms, W1, b1, W2, b2)
